# baseline (speedup 1.0000x reference)
import functools

import jax
import jax.numpy as jnp
from jax import lax
from jax.experimental import pallas as pl
from jax.experimental.pallas import tpu as pltpu
from jax.experimental.pallas import tpu_sc as plsc

N = 10000
E = 160000
D = 128
K = 5
NK = K * K

NW = 32
NPAD = 10240
RPW = NPAD // NW
BLK = 4000
NBLK = E // BLK
NEG = -3.0e38

_ER = 1250
_NB = 400
_NG = N // _NB


def _trans_body(x_ref, w_ref, out_ref):
    xb = x_ref[...]
    for k in range(NK):
        out_ref[:, k, :] = jnp.dot(xb, w_ref[k], preferred_element_type=jnp.float32)


def _trans_call(x, W):
    return pl.pallas_call(
        _trans_body,
        grid=(_NG,),
        in_specs=[
            pl.BlockSpec((_NB, D), lambda i: (i, 0)),
            pl.BlockSpec((NK, D, D), lambda i: (0, 0, 0)),
        ],
        out_specs=pl.BlockSpec((_NB, NK, D), lambda i: (i, 0, 0)),
        out_shape=jax.ShapeDtypeStruct((N, NK, D), jnp.float32),
    )(x, W)


def _prep_body(src_ref, ea0_ref, ea1_ref, *out_refs):
    src = src_ref[...]
    v0 = ea0_ref[...] * (K - 1.0)
    v1 = ea1_ref[...] * (K - 1.0)
    f0 = jnp.floor(v0)
    f1 = jnp.floor(v1)
    fr0 = v0 - f0
    fr1 = v1 - f1
    b0 = jnp.clip(f0.astype(jnp.int32), 0, K - 2)
    b1 = jnp.clip(f1.astype(jnp.int32), 0, K - 2)
    base = src * NK + b1 * K + b0
    j = 0
    for s0 in (0, 1):
        for s1 in (0, 1):
            out_refs[j][...] = base + (s1 * K + s0)
            w0 = fr0 if s0 else 1.0 - fr0
            w1 = fr1 if s1 else 1.0 - fr1
            out_refs[4 + j][...] = lax.bitcast_convert_type(w0 * w1, jnp.int32)
            j += 1


def _prep_call(src2d, ea02d, ea12d):
    outs = pl.pallas_call(
        _prep_body,
        out_shape=[jax.ShapeDtypeStruct((_ER, D), jnp.int32)] * 8,
    )(src2d, ea02d, ea12d)
    return jnp.stack(outs, axis=-1).reshape(E * 8)


def _final_body(agg_ref, x_ref, root_ref, bias_ref, out_ref, *, relu):
    a = agg_ref[...]
    a = jnp.where(a < -1.0e37, 0.0, a)
    r = jnp.dot(x_ref[...], root_ref[...], preferred_element_type=jnp.float32)
    o = a + r + bias_ref[...]
    if relu:
        o = jnp.maximum(o, 0.0)
    out_ref[...] = o


def _final_call(agg, x, root, bias, relu):
    return pl.pallas_call(
        functools.partial(_final_body, relu=relu),
        grid=(_NG,),
        in_specs=[
            pl.BlockSpec((_NB, D), lambda i: (i, 0)),
            pl.BlockSpec((_NB, D), lambda i: (i, 0)),
            pl.BlockSpec((D, D), lambda i: (0, 0)),
            pl.BlockSpec((1, D), lambda i: (0, 0)),
        ],
        out_specs=pl.BlockSpec((_NB, D), lambda i: (i, 0)),
        out_shape=jax.ShapeDtypeStruct((N, D), jnp.float32),
    )(agg, x, root, bias.reshape(1, D))


_MESH = dict(core_axis_name="c", subcore_axis_name="s", num_cores=2,
             num_subcores=16)


@functools.partial(
    pl.kernel,
    out_type=jax.ShapeDtypeStruct((NPAD * D,), jnp.float32),
    mesh=plsc.VectorSubcoreMesh(**_MESH),
    compiler_params=pltpu.CompilerParams(needs_layout_passes=False),
    scratch_types=[
        pltpu.VMEM((RPW * D,), jnp.float32),
        pltpu.VMEM((BLK,), jnp.int32),
        pltpu.VMEM((BLK * 8,), jnp.int32),
        pltpu.VMEM((BLK + 160,), jnp.int32),
        pltpu.VMEM((BLK + 160,), jnp.int32),
        pltpu.VMEM((2 * 128,), jnp.int32),
        pltpu.VMEM((2 * 128, D), jnp.float32),
        pltpu.SemaphoreType.DMA,
        pltpu.SemaphoreType.DMA,
    ],
)
def _sc_agg(trans_hbm, meta_hbm, dst_hbm, out_hbm, acc, dstb, metab, lidx,
            mrow, tidx, grows, sem, msem):
    cid = lax.axis_index("c")
    sid = lax.axis_index("s")
    wid = sid * 2 + cid
    lo = wid * RPW
    lanes = lax.iota(jnp.int32, 16)
    neg = jnp.full((16,), NEG, jnp.float32)

    def init_body(i, _):
        acc[pl.ds(i * 16, 16)] = neg
        return 0

    lax.fori_loop(0, RPW * D // 16, init_body, 0)

    def block_body(blk, _):
        ebase = blk * BLK
        mdma = pltpu.async_copy(meta_hbm.at[pl.ds(ebase * 8, BLK * 8)],
                                metab, msem)
        pltpu.sync_copy(dst_hbm.at[pl.ds(ebase, BLK)], dstb)

        def scan_body(c4, cnt):
            for u in range(2):
                c = c4 * 2 + u
                d = dstb[pl.ds(c * 16, 16)]
                msk = (d >= lo) & (d < lo + RPW)
                mi = jnp.where(msk, 1, 0)
                cs = plsc.cumsum(mi)
                pos = cnt + cs - mi
                plsc.store_scatter(mrow, [pos], d - lo, mask=msk)
                plsc.store_scatter(lidx, [pos], c * 16 + lanes, mask=msk)
                cnt = cnt + jnp.max(plsc.all_reduce_population_count(msk))
            return cnt

        m = lax.fori_loop(0, BLK // 32, scan_body, 0)
        pad = jnp.full((16,), (wid * 101) % BLK, jnp.int32)
        for z in range(8):
            lidx[pl.ds(m + z * 16, 16)] = pad
        mdma.wait()

        ngrp = (m + 31) // 32

        def build_tidx(g):
            pb = (g % 2) * 128
            base_g = g * 32
            for q in range(8):
                lrow = plsc.load_gather(
                    lidx, [base_g + q * 4 + lanes // 4])
                tidx[pl.ds(pb + q * 16, 16)] = plsc.load_gather(
                    metab, [lrow * 8 + (lanes % 4)])
            pltpu.async_copy(trans_hbm.at[tidx.at[pl.ds(pb, 128)]],
                             grows.at[pl.ds(pb, 128)], sem)

        def wait_gather():
            pltpu.make_async_copy(trans_hbm.at[tidx.at[pl.ds(0, 128)]],
                                  grows.at[pl.ds(0, 128)], sem).wait()

        def process(g):
            pb = (g % 2) * 128
            base_g = g * 32
            cnt_e = jnp.clip(m - base_g, 0, 32)

            def edge_body(i, _):
                lv = lidx[pl.ds(base_g + i, 16)]
                rv = mrow[pl.ds(base_g + i, 16)]
                r = rv[0]
                bv = plsc.load_gather(
                    metab, [lv[0] * 8 + 4 + (lanes & 3)])
                bf = lax.bitcast_convert_type(bv, jnp.float32)
                b0 = lax.broadcast_in_dim(bf[0], (16,), ())
                b1 = lax.broadcast_in_dim(bf[1], (16,), ())
                b2 = lax.broadcast_in_dim(bf[2], (16,), ())
                b3 = lax.broadcast_in_dim(bf[3], (16,), ())
                for c8 in range(8):
                    sl = pl.ds(c8 * 16, 16)
                    g0 = grows[pb + i * 4 + 0, sl]
                    g1 = grows[pb + i * 4 + 1, sl]
                    g2 = grows[pb + i * 4 + 2, sl]
                    g3 = grows[pb + i * 4 + 3, sl]
                    msg = b0 * g0 + b1 * g1 + b2 * g2 + b3 * g3
                    asl = pl.ds(r * D + c8 * 16, 16)
                    acc[asl] = jnp.maximum(acc[asl], msg)
                return 0

            lax.fori_loop(0, cnt_e, edge_body, 0)

        @pl.when(ngrp > 0)
        def _pipe():
            build_tidx(0)

            def grp_body(g, _):
                build_tidx(g)
                wait_gather()
                process(g - 1)
                return 0

            lax.fori_loop(1, ngrp, grp_body, 0)
            wait_gather()
            process(ngrp - 1)

        return 0

    lax.fori_loop(0, NBLK, block_body, 0)
    pltpu.sync_copy(acc, out_hbm.at[pl.ds(wid * RPW * D, RPW * D)])


def kernel(x, edge_index, edge_attr, W1, root1, bias1, W2, root2, bias2):
    src = edge_index[0]
    dst = edge_index[1]
    meta = _prep_call(src.reshape(_ER, D),
                      edge_attr[:, 0].reshape(_ER, D),
                      edge_attr[:, 1].reshape(_ER, D))

    t1 = _trans_call(x, W1).reshape(N * NK, D)
    agg1 = _sc_agg(t1, meta, dst).reshape(NPAD, D)[:N]
    h = _final_call(agg1, x, root1, bias1, relu=True)

    t2 = _trans_call(h, W2).reshape(N * NK, D)
    agg2 = _sc_agg(t2, meta, dst).reshape(NPAD, D)[:N]
    out = _final_call(agg2, h, root2, bias2, relu=False)
    return out


# speedup vs baseline: 1.0366x; 1.0366x over previous
import functools

import jax
import jax.numpy as jnp
from jax import lax
from jax.experimental import pallas as pl
from jax.experimental.pallas import tpu as pltpu
from jax.experimental.pallas import tpu_sc as plsc

N = 10000
E = 160000
D = 128
K = 5
NK = K * K

NW = 32
NPAD = 10240
RPW = NPAD // NW
BLK = 4000
NBLK = E // BLK
NEG = -3.0e38

_ER = 1250
_NB = 400
_NG = N // _NB


def _trans_body(x_ref, w_ref, out_ref):
    xb = x_ref[...]
    for k in range(NK):
        out_ref[:, k, :] = jnp.dot(xb, w_ref[k], preferred_element_type=jnp.float32)


def _trans_call(x, W):
    return pl.pallas_call(
        _trans_body,
        grid=(_NG,),
        in_specs=[
            pl.BlockSpec((_NB, D), lambda i: (i, 0)),
            pl.BlockSpec((NK, D, D), lambda i: (0, 0, 0)),
        ],
        out_specs=pl.BlockSpec((_NB, NK, D), lambda i: (i, 0, 0)),
        out_shape=jax.ShapeDtypeStruct((N, NK, D), jnp.float32),
    )(x, W)


def _prep_body(src_ref, ea0_ref, ea1_ref, *out_refs):
    src = src_ref[...]
    v0 = ea0_ref[...] * (K - 1.0)
    v1 = ea1_ref[...] * (K - 1.0)
    f0 = jnp.floor(v0)
    f1 = jnp.floor(v1)
    fr0 = v0 - f0
    fr1 = v1 - f1
    b0 = jnp.clip(f0.astype(jnp.int32), 0, K - 2)
    b1 = jnp.clip(f1.astype(jnp.int32), 0, K - 2)
    base = src * NK + b1 * K + b0
    j = 0
    for s0 in (0, 1):
        for s1 in (0, 1):
            out_refs[j][...] = base + (s1 * K + s0)
            w0 = fr0 if s0 else 1.0 - fr0
            w1 = fr1 if s1 else 1.0 - fr1
            out_refs[4 + j][...] = lax.bitcast_convert_type(w0 * w1, jnp.int32)
            j += 1


def _prep_call(src2d, ea02d, ea12d):
    outs = pl.pallas_call(
        _prep_body,
        out_shape=[jax.ShapeDtypeStruct((_ER, D), jnp.int32)] * 8,
    )(src2d, ea02d, ea12d)
    return jnp.stack(outs, axis=-1).reshape(E * 8)


def _final_body(agg_ref, x_ref, root_ref, bias_ref, out_ref, *, relu):
    a = agg_ref[...]
    a = jnp.where(a < -1.0e37, 0.0, a)
    r = jnp.dot(x_ref[...], root_ref[...], preferred_element_type=jnp.float32)
    o = a + r + bias_ref[...]
    if relu:
        o = jnp.maximum(o, 0.0)
    out_ref[...] = o


def _final_call(agg, x, root, bias, relu):
    return pl.pallas_call(
        functools.partial(_final_body, relu=relu),
        grid=(_NG,),
        in_specs=[
            pl.BlockSpec((_NB, D), lambda i: (i, 0)),
            pl.BlockSpec((_NB, D), lambda i: (i, 0)),
            pl.BlockSpec((D, D), lambda i: (0, 0)),
            pl.BlockSpec((1, D), lambda i: (0, 0)),
        ],
        out_specs=pl.BlockSpec((_NB, D), lambda i: (i, 0)),
        out_shape=jax.ShapeDtypeStruct((N, D), jnp.float32),
    )(agg, x, root, bias.reshape(1, D))


_MESH = dict(core_axis_name="c", subcore_axis_name="s", num_cores=2,
             num_subcores=16)


@functools.partial(
    pl.kernel,
    out_type=jax.ShapeDtypeStruct((NPAD * D,), jnp.float32),
    mesh=plsc.VectorSubcoreMesh(**_MESH),
    compiler_params=pltpu.CompilerParams(needs_layout_passes=False),
    scratch_types=[
        pltpu.VMEM(((RPW + 1) * D,), jnp.float32),
        pltpu.VMEM((BLK,), jnp.int32),
        pltpu.VMEM((BLK * 8,), jnp.int32),
        pltpu.VMEM((BLK + 160,), jnp.int32),
        pltpu.VMEM((BLK + 160,), jnp.int32),
        pltpu.VMEM((2 * 128,), jnp.int32),
        pltpu.VMEM((2 * 128, D), jnp.float32),
        pltpu.SemaphoreType.DMA,
        pltpu.SemaphoreType.DMA,
    ],
)
def _sc_agg(trans_hbm, meta_hbm, dst_hbm, out_hbm, acc, dstb, metab, lidx,
            mrow, tidx, grows, sem, msem):
    cid = lax.axis_index("c")
    sid = lax.axis_index("s")
    wid = sid * 2 + cid
    lo = wid * RPW
    lanes = lax.iota(jnp.int32, 16)
    neg = jnp.full((16,), NEG, jnp.float32)

    def init_body(i, _):
        acc[pl.ds(i * 16, 16)] = neg
        return 0

    lax.fori_loop(0, (RPW + 1) * D // 16, init_body, 0)

    def block_body(blk, _):
        ebase = blk * BLK
        mdma = pltpu.async_copy(meta_hbm.at[pl.ds(ebase * 8, BLK * 8)],
                                metab, msem)
        pltpu.sync_copy(dst_hbm.at[pl.ds(ebase, BLK)], dstb)

        def scan_body(c4, cnt):
            for u in range(2):
                c = c4 * 2 + u
                d = dstb[pl.ds(c * 16, 16)]
                msk = (d >= lo) & (d < lo + RPW)
                mi = jnp.where(msk, 1, 0)
                cs = plsc.cumsum(mi)
                pos = cnt + cs - mi
                plsc.store_scatter(mrow, [pos], d - lo, mask=msk)
                plsc.store_scatter(lidx, [pos], c * 16 + lanes, mask=msk)
                cnt = cnt + jnp.max(plsc.all_reduce_population_count(msk))
            return cnt

        m = lax.fori_loop(0, BLK // 32, scan_body, 0)
        pad = jnp.full((16,), (wid * 101) % BLK, jnp.int32)
        padr = jnp.full((16,), RPW, jnp.int32)
        mrow[pl.ds(m, 16)] = padr
        for z in range(8):
            lidx[pl.ds(m + z * 16, 16)] = pad
        mdma.wait()

        ngrp = (m + 31) // 32

        def build_tidx(g):
            pb = (g % 2) * 128
            base_g = g * 32
            for q in range(8):
                lrow = plsc.load_gather(
                    lidx, [base_g + q * 4 + lanes // 4])
                tidx[pl.ds(pb + q * 16, 16)] = plsc.load_gather(
                    metab, [lrow * 8 + (lanes % 4)])
            pltpu.async_copy(trans_hbm.at[tidx.at[pl.ds(pb, 128)]],
                             grows.at[pl.ds(pb, 128)], sem)

        def wait_gather():
            pltpu.make_async_copy(trans_hbm.at[tidx.at[pl.ds(0, 128)]],
                                  grows.at[pl.ds(0, 128)], sem).wait()

        def process(g):
            pb = (g % 2) * 128
            base_g = g * 32
            cnt_e = jnp.clip(m - base_g, 0, 32)

            def edge_body(p, _):
                i0 = p * 2
                lv = lidx[pl.ds(base_g + i0, 16)]
                rv = mrow[pl.ds(base_g + i0, 16)]
                ra = rv[0]
                rb = rv[1]
                bva = plsc.load_gather(
                    metab, [lv[0] * 8 + 4 + (lanes & 3)])
                bvb = plsc.load_gather(
                    metab, [lv[1] * 8 + 4 + (lanes & 3)])
                bfa = lax.bitcast_convert_type(bva, jnp.float32)
                bfb = lax.bitcast_convert_type(bvb, jnp.float32)
                a0 = lax.broadcast_in_dim(bfa[0], (16,), ())
                a1 = lax.broadcast_in_dim(bfa[1], (16,), ())
                a2 = lax.broadcast_in_dim(bfa[2], (16,), ())
                a3 = lax.broadcast_in_dim(bfa[3], (16,), ())
                b0 = lax.broadcast_in_dim(bfb[0], (16,), ())
                b1 = lax.broadcast_in_dim(bfb[1], (16,), ())
                b2 = lax.broadcast_in_dim(bfb[2], (16,), ())
                b3 = lax.broadcast_in_dim(bfb[3], (16,), ())
                for c8 in range(8):
                    sl = pl.ds(c8 * 16, 16)
                    ga0 = grows[pb + i0 * 4 + 0, sl]
                    ga1 = grows[pb + i0 * 4 + 1, sl]
                    ga2 = grows[pb + i0 * 4 + 2, sl]
                    ga3 = grows[pb + i0 * 4 + 3, sl]
                    msga = a0 * ga0 + a1 * ga1 + a2 * ga2 + a3 * ga3
                    asla = pl.ds(ra * D + c8 * 16, 16)
                    acc[asla] = jnp.maximum(acc[asla], msga)
                for c8 in range(8):
                    sl = pl.ds(c8 * 16, 16)
                    gb0 = grows[pb + i0 * 4 + 4, sl]
                    gb1 = grows[pb + i0 * 4 + 5, sl]
                    gb2 = grows[pb + i0 * 4 + 6, sl]
                    gb3 = grows[pb + i0 * 4 + 7, sl]
                    msgb = b0 * gb0 + b1 * gb1 + b2 * gb2 + b3 * gb3
                    aslb = pl.ds(rb * D + c8 * 16, 16)
                    acc[aslb] = jnp.maximum(acc[aslb], msgb)
                return 0

            lax.fori_loop(0, (cnt_e + 1) // 2, edge_body, 0)

        @pl.when(ngrp > 0)
        def _pipe():
            build_tidx(0)

            def grp_body(g, _):
                build_tidx(g)
                wait_gather()
                process(g - 1)
                return 0

            lax.fori_loop(1, ngrp, grp_body, 0)
            wait_gather()
            process(ngrp - 1)

        return 0

    lax.fori_loop(0, NBLK, block_body, 0)
    pltpu.sync_copy(acc.at[pl.ds(0, RPW * D)],
                    out_hbm.at[pl.ds(wid * RPW * D, RPW * D)])


def kernel(x, edge_index, edge_attr, W1, root1, bias1, W2, root2, bias2):
    src = edge_index[0]
    dst = edge_index[1]
    meta = _prep_call(src.reshape(_ER, D),
                      edge_attr[:, 0].reshape(_ER, D),
                      edge_attr[:, 1].reshape(_ER, D))

    t1 = _trans_call(x, W1).reshape(N * NK, D)
    agg1 = _sc_agg(t1, meta, dst).reshape(NPAD, D)[:N]
    h = _final_call(agg1, x, root1, bias1, relu=True)

    t2 = _trans_call(h, W2).reshape(N * NK, D)
    agg2 = _sc_agg(t2, meta, dst).reshape(NPAD, D)[:N]
    out = _final_call(agg2, h, root2, bias2, relu=False)
    return out


# speedup vs baseline: 1.0913x; 1.0527x over previous
import functools

import jax
import jax.numpy as jnp
from jax import lax
from jax.experimental import pallas as pl
from jax.experimental.pallas import tpu as pltpu
from jax.experimental.pallas import tpu_sc as plsc

N = 10000
E = 160000
D = 128
K = 5
NK = K * K

NW = 32
NPAD = 10240
RPW = NPAD // NW
BLK = 4000
NBLK = E // BLK
NEG = -3.0e38

_ER = 1250
_NB = 1000
_NG = N // _NB


def _trans_body(x_ref, w_ref, out_ref):
    xb = x_ref[...]
    for k in range(NK):
        out_ref[:, k, :] = jnp.dot(xb, w_ref[k], preferred_element_type=jnp.float32)


def _trans_call(x, W):
    return pl.pallas_call(
        _trans_body,
        grid=(_NG,),
        in_specs=[
            pl.BlockSpec((_NB, D), lambda i: (i, 0)),
            pl.BlockSpec((NK, D, D), lambda i: (0, 0, 0)),
        ],
        out_specs=pl.BlockSpec((_NB, NK, D), lambda i: (i, 0, 0)),
        out_shape=jax.ShapeDtypeStruct((N, NK, D), jnp.float32),
    )(x, W)


def _prep_body(src_ref, ea0_ref, ea1_ref, *out_refs):
    src = src_ref[...]
    v0 = ea0_ref[...] * (K - 1.0)
    v1 = ea1_ref[...] * (K - 1.0)
    f0 = jnp.floor(v0)
    f1 = jnp.floor(v1)
    fr0 = v0 - f0
    fr1 = v1 - f1
    b0 = jnp.clip(f0.astype(jnp.int32), 0, K - 2)
    b1 = jnp.clip(f1.astype(jnp.int32), 0, K - 2)
    base = src * NK + b1 * K + b0
    j = 0
    for s0 in (0, 1):
        for s1 in (0, 1):
            out_refs[j][...] = base + (s1 * K + s0)
            w0 = fr0 if s0 else 1.0 - fr0
            w1 = fr1 if s1 else 1.0 - fr1
            out_refs[4 + j][...] = lax.bitcast_convert_type(w0 * w1, jnp.int32)
            j += 1


def _prep_call(src2d, ea02d, ea12d):
    outs = pl.pallas_call(
        _prep_body,
        out_shape=[jax.ShapeDtypeStruct((_ER, D), jnp.int32)] * 8,
    )(src2d, ea02d, ea12d)
    return jnp.stack(outs, axis=-1).reshape(E * 8)


def _final_body(agg_ref, x_ref, root_ref, bias_ref, out_ref, *, relu):
    a = agg_ref[...]
    a = jnp.where(a < -1.0e37, 0.0, a)
    r = jnp.dot(x_ref[...], root_ref[...], preferred_element_type=jnp.float32)
    o = a + r + bias_ref[...]
    if relu:
        o = jnp.maximum(o, 0.0)
    out_ref[...] = o


def _final_call(agg, x, root, bias, relu):
    return pl.pallas_call(
        functools.partial(_final_body, relu=relu),
        grid=(_NG,),
        in_specs=[
            pl.BlockSpec((_NB, D), lambda i: (i, 0)),
            pl.BlockSpec((_NB, D), lambda i: (i, 0)),
            pl.BlockSpec((D, D), lambda i: (0, 0)),
            pl.BlockSpec((1, D), lambda i: (0, 0)),
        ],
        out_specs=pl.BlockSpec((_NB, D), lambda i: (i, 0)),
        out_shape=jax.ShapeDtypeStruct((N, D), jnp.float32),
    )(agg, x, root, bias.reshape(1, D))


_MESH = dict(core_axis_name="c", subcore_axis_name="s", num_cores=2,
             num_subcores=16)


@functools.partial(
    pl.kernel,
    out_type=jax.ShapeDtypeStruct((NPAD * D,), jnp.float32),
    mesh=plsc.VectorSubcoreMesh(**_MESH),
    compiler_params=pltpu.CompilerParams(needs_layout_passes=False),
    scratch_types=[
        pltpu.VMEM(((RPW + 1) * D,), jnp.float32),
        pltpu.VMEM((2 * BLK,), jnp.int32),
        pltpu.VMEM((BLK * 8,), jnp.int32),
        pltpu.VMEM((BLK + 160,), jnp.int32),
        pltpu.VMEM((BLK + 160,), jnp.int32),
        pltpu.VMEM((2 * 128,), jnp.int32),
        pltpu.VMEM((2 * 128, D), jnp.float32),
        pltpu.SemaphoreType.DMA,
        pltpu.SemaphoreType.DMA,
        pltpu.SemaphoreType.DMA,
    ],
)
def _sc_agg(trans_hbm, meta_hbm, dst_hbm, out_hbm, acc, dstb, metab, lidx,
            mrow, tidx, grows, sem, msem, dsem):
    cid = lax.axis_index("c")
    sid = lax.axis_index("s")
    wid = sid * 2 + cid
    lo = wid * RPW
    lanes = lax.iota(jnp.int32, 16)
    neg = jnp.full((16,), NEG, jnp.float32)

    def init_body(i, _):
        acc[pl.ds(i * 16, 16)] = neg
        return 0

    lax.fori_loop(0, (RPW + 1) * D // 16, init_body, 0)

    pltpu.sync_copy(dst_hbm.at[pl.ds(0, BLK)], dstb.at[pl.ds(0, BLK)])

    def block_body(blk, _):
        ebase = blk * BLK
        dbo = (blk % 2) * BLK
        mdma = pltpu.async_copy(meta_hbm.at[pl.ds(ebase * 8, BLK * 8)],
                                metab, msem)

        @pl.when(blk + 1 < NBLK)
        def _pf():
            pltpu.async_copy(
                dst_hbm.at[pl.ds(ebase + BLK, BLK)],
                dstb.at[pl.ds(BLK - dbo, BLK)], dsem)

        def scan_body(c4, cnt):
            for u in range(2):
                c = c4 * 2 + u
                d = dstb[pl.ds(dbo + c * 16, 16)]
                msk = (d >= lo) & (d < lo + RPW)
                mi = jnp.where(msk, 1, 0)
                cs = plsc.cumsum(mi)
                pos = cnt + cs - mi
                plsc.store_scatter(mrow, [pos], d - lo, mask=msk)
                plsc.store_scatter(lidx, [pos], c * 16 + lanes, mask=msk)
                cnt = cnt + jnp.max(plsc.all_reduce_population_count(msk))
            return cnt

        m = lax.fori_loop(0, BLK // 32, scan_body, 0)
        pad = jnp.full((16,), (wid * 101) % BLK, jnp.int32)
        padr = jnp.full((16,), RPW, jnp.int32)
        mrow[pl.ds(m, 16)] = padr
        for z in range(8):
            lidx[pl.ds(m + z * 16, 16)] = pad
        mdma.wait()

        ngrp = (m + 31) // 32

        def build_tidx(g):
            pb = (g % 2) * 128
            base_g = g * 32
            for q in range(8):
                lrow = plsc.load_gather(
                    lidx, [base_g + q * 4 + lanes // 4])
                tidx[pl.ds(pb + q * 16, 16)] = plsc.load_gather(
                    metab, [lrow * 8 + (lanes % 4)])
            pltpu.async_copy(trans_hbm.at[tidx.at[pl.ds(pb, 128)]],
                             grows.at[pl.ds(pb, 128)], sem)

        def wait_gather():
            pltpu.make_async_copy(trans_hbm.at[tidx.at[pl.ds(0, 128)]],
                                  grows.at[pl.ds(0, 128)], sem).wait()

        def process(g):
            pb = (g % 2) * 128
            base_g = g * 32
            cnt_e = jnp.clip(m - base_g, 0, 32)

            def edge_body(p, _):
                i0 = p * 2
                lv = lidx[pl.ds(base_g + i0, 16)]
                rv = mrow[pl.ds(base_g + i0, 16)]
                ra = rv[0]
                rb = rv[1]
                bva = plsc.load_gather(
                    metab, [lv[0] * 8 + 4 + (lanes & 3)])
                bvb = plsc.load_gather(
                    metab, [lv[1] * 8 + 4 + (lanes & 3)])
                bfa = lax.bitcast_convert_type(bva, jnp.float32)
                bfb = lax.bitcast_convert_type(bvb, jnp.float32)
                a0 = lax.broadcast_in_dim(bfa[0], (16,), ())
                a1 = lax.broadcast_in_dim(bfa[1], (16,), ())
                a2 = lax.broadcast_in_dim(bfa[2], (16,), ())
                a3 = lax.broadcast_in_dim(bfa[3], (16,), ())
                b0 = lax.broadcast_in_dim(bfb[0], (16,), ())
                b1 = lax.broadcast_in_dim(bfb[1], (16,), ())
                b2 = lax.broadcast_in_dim(bfb[2], (16,), ())
                b3 = lax.broadcast_in_dim(bfb[3], (16,), ())
                for c8 in range(8):
                    sl = pl.ds(c8 * 16, 16)
                    ga0 = grows[pb + i0 * 4 + 0, sl]
                    ga1 = grows[pb + i0 * 4 + 1, sl]
                    ga2 = grows[pb + i0 * 4 + 2, sl]
                    ga3 = grows[pb + i0 * 4 + 3, sl]
                    msga = a0 * ga0 + a1 * ga1 + a2 * ga2 + a3 * ga3
                    asla = pl.ds(ra * D + c8 * 16, 16)
                    acc[asla] = jnp.maximum(acc[asla], msga)
                for c8 in range(8):
                    sl = pl.ds(c8 * 16, 16)
                    gb0 = grows[pb + i0 * 4 + 4, sl]
                    gb1 = grows[pb + i0 * 4 + 5, sl]
                    gb2 = grows[pb + i0 * 4 + 6, sl]
                    gb3 = grows[pb + i0 * 4 + 7, sl]
                    msgb = b0 * gb0 + b1 * gb1 + b2 * gb2 + b3 * gb3
                    aslb = pl.ds(rb * D + c8 * 16, 16)
                    acc[aslb] = jnp.maximum(acc[aslb], msgb)
                return 0

            lax.fori_loop(0, (cnt_e + 1) // 2, edge_body, 0)

        @pl.when(ngrp > 0)
        def _pipe():
            build_tidx(0)

            def grp_body(g, _):
                build_tidx(g)
                wait_gather()
                process(g - 1)
                return 0

            lax.fori_loop(1, ngrp, grp_body, 0)
            wait_gather()
            process(ngrp - 1)

        @pl.when(blk + 1 < NBLK)
        def _pfw():
            pltpu.make_async_copy(
                dst_hbm.at[pl.ds(0, BLK)],
                dstb.at[pl.ds(0, BLK)], dsem).wait()
        return 0

    lax.fori_loop(0, NBLK, block_body, 0)
    pltpu.sync_copy(acc.at[pl.ds(0, RPW * D)],
                    out_hbm.at[pl.ds(wid * RPW * D, RPW * D)])


def kernel(x, edge_index, edge_attr, W1, root1, bias1, W2, root2, bias2):
    src = edge_index[0]
    dst = edge_index[1]
    meta = _prep_call(src.reshape(_ER, D),
                      edge_attr[:, 0].reshape(_ER, D),
                      edge_attr[:, 1].reshape(_ER, D))

    t1 = _trans_call(x, W1).reshape(N * NK, D)
    agg1 = _sc_agg(t1, meta, dst).reshape(NPAD, D)[:N]
    h = _final_call(agg1, x, root1, bias1, relu=True)

    t2 = _trans_call(h, W2).reshape(N * NK, D)
    agg2 = _sc_agg(t2, meta, dst).reshape(NPAD, D)[:N]
    out = _final_call(agg2, h, root2, bias2, relu=False)
    return out
